# baseline (speedup 1.0000x reference)
import dataclasses

import jax
import jax.numpy as jnp
from jax import lax
from jax.experimental import pallas as pl
from jax.experimental.pallas import tpu as pltpu
from jax.experimental.pallas import tpu_sc as plsc

B, T, N = 16, 2048, 16
H = 64
NF = 7
NAA = 21
HE = H - NF
NFC = N * NF

NC, NS, L = 2, 16, 16
TCH = 128
NCHUNK = T // (NC * TCH)
HG = 8
NTV = TCH // L
TS = TCH + 1


def _sc_kernel(x_hbm, f_hbm, e_hbm, out_hbm,
               xv, xt, fv, ft, ev, ov0, ov1, sem0, sem1):
    b = lax.axis_index("s")
    th = lax.axis_index("c")

    pltpu.sync_copy(e_hbm, ev)

    i16 = lax.iota(jnp.int32, L)
    i_ts = i16 * TS
    full = i16 >= 0

    @pl.loop(0, NCHUNK)
    def _chunk(ck):
        t0 = pl.multiple_of(th * (NCHUNK * TCH) + ck * TCH, TCH)
        pltpu.sync_copy(x_hbm.at[b, pl.ds(t0 * N, TCH * N)], xv)
        pltpu.sync_copy(f_hbm.at[b, pl.ds(t0 * NFC, TCH * NFC)], fv)

        @plsc.parallel_loop(0, TCH, unroll=8)
        def _rp(t):
            code = xv[pl.ds(t * N, L)]
            plsc.store_scatter(xt, [i_ts + t], code)
            for j in range(NFC // L):
                vals = fv[pl.ds(t * NFC + j * L, L)]
                plsc.store_scatter(ft, [i_ts + (j * L * TS + t)], vals)

        inflight = [None, None]
        for hg in range(H // HG):
            ov = ov0 if hg % 2 == 0 else ov1
            sem = sem0 if hg % 2 == 0 else sem1
            if inflight[hg % 2] is not None:
                inflight[hg % 2].wait()

            @plsc.parallel_loop(0, N * NTV, unroll=16)
            def _q(q, ov=ov, hg=hg):
                n = q // NTV
                tv = q % NTV
                aa = xt[pl.ds(n * TS + tv * L, L)]
                lidx = aa * (HE * L) + i16
                toff = tv * L
                for hl in range(HG):
                    h = hg * HG + hl
                    if h < HE:
                        val = plsc.load_gather(ev, [lidx + h * L])
                    else:
                        c = n * NF + (h - HE)
                        val = ft[pl.ds(c * TS + tv * L, L)]
                    ov[hl, n, pl.ds(toff, L)] = val

            inflight[hg % 2] = pltpu.async_copy(
                ov, out_hbm.at[b, pl.ds(hg * HG, HG), :, pl.ds(t0, TCH)], sem)
        inflight[0].wait()
        inflight[1].wait()


def kernel(X, features, emb):
    x2 = X.astype(jnp.int32).reshape(B, T * N)
    f2 = features.reshape(B, T * NFC)
    lut = jnp.broadcast_to(emb.reshape(-1)[:, None], (NAA * HE, L)).reshape(-1)

    cp = pltpu.CompilerParams()
    if "needs_layout_passes" in pltpu.CompilerParams.__dataclass_fields__:
        cp = dataclasses.replace(cp, needs_layout_passes=False)
    mesh = plsc.VectorSubcoreMesh(core_axis_name="c", subcore_axis_name="s")
    k = pl.kernel(
        _sc_kernel,
        out_type=jax.ShapeDtypeStruct((B, H, N, T), jnp.float32),
        mesh=mesh,
        compiler_params=cp,
        scratch_types=[
            pltpu.VMEM((TCH * N,), jnp.int32),
            pltpu.VMEM((N * TS,), jnp.int32),
            pltpu.VMEM((TCH * NFC,), jnp.float32),
            pltpu.VMEM((NFC * TS,), jnp.float32),
            pltpu.VMEM((NAA * HE * L,), jnp.float32),
            pltpu.VMEM((HG, N, TCH), jnp.float32),
            pltpu.VMEM((HG, N, TCH), jnp.float32),
            pltpu.SemaphoreType.DMA,
            pltpu.SemaphoreType.DMA,
        ],
    )
    return k(x2, f2, lut)


# speedup vs baseline: 1.0304x; 1.0304x over previous
import dataclasses

import jax
import jax.numpy as jnp
from jax import lax
from jax.experimental import pallas as pl
from jax.experimental.pallas import tpu as pltpu
from jax.experimental.pallas import tpu_sc as plsc

B, T, N = 16, 2048, 16
H = 64
NF = 7
NAA = 21
HE = H - NF
NFC = N * NF

NC, NS, L = 2, 16, 16
TCH = 128
NCHUNK = T // (NC * TCH)
HG = 8
NTV = TCH // L
TS = TCH + 1


def _sc_kernel(x_hbm, f_hbm, e_hbm, out_hbm,
               xv, xt, fv, ft, ev, ov0, ov1, sem0, sem1):
    b = lax.axis_index("s")
    th = lax.axis_index("c")

    pltpu.sync_copy(e_hbm, ev)

    i16 = lax.iota(jnp.int32, L)
    i_ts = i16 * TS
    full = i16 >= 0

    @pl.loop(0, NCHUNK)
    def _chunk(ck):
        t0 = pl.multiple_of(th * (NCHUNK * TCH) + ck * TCH, TCH)
        pltpu.sync_copy(x_hbm.at[b, pl.ds(t0 * N, TCH * N)], xv)
        pltpu.sync_copy(f_hbm.at[b, pl.ds(t0 * NFC, TCH * NFC)], fv)

        @plsc.parallel_loop(0, TCH, unroll=4)
        def _rp(t):
            code = xv[pl.ds(t * N, L)]
            plsc.store_scatter(xt, [i_ts + t], code)
            for j in range(NFC // L):
                vals = fv[pl.ds(t * NFC + j * L, L)]
                plsc.store_scatter(ft, [i_ts + (j * L * TS + t)], vals)

        inflight = [None, None]
        for hg in range(H // HG):
            ov = ov0 if hg % 2 == 0 else ov1
            sem = sem0 if hg % 2 == 0 else sem1
            if inflight[hg % 2] is not None:
                inflight[hg % 2].wait()

            @plsc.parallel_loop(0, N * NTV, unroll=8)
            def _q(q, ov=ov, hg=hg):
                n = q // NTV
                tv = q % NTV
                aa = xt[pl.ds(n * TS + tv * L, L)]
                lidx = aa * (HE * L) + i16
                toff = tv * L
                for hl in range(HG):
                    h = hg * HG + hl
                    if h < HE:
                        val = plsc.load_gather(ev, [lidx + h * L])
                    else:
                        c = n * NF + (h - HE)
                        val = ft[pl.ds(c * TS + tv * L, L)]
                    ov[hl, n, pl.ds(toff, L)] = val

            inflight[hg % 2] = pltpu.async_copy(
                ov, out_hbm.at[b, pl.ds(hg * HG, HG), :, pl.ds(t0, TCH)], sem)
        inflight[0].wait()
        inflight[1].wait()


def kernel(X, features, emb):
    x2 = X.astype(jnp.int32).reshape(B, T * N)
    f2 = features.reshape(B, T * NFC)
    lut = jnp.broadcast_to(emb.reshape(-1)[:, None], (NAA * HE, L)).reshape(-1)

    cp = pltpu.CompilerParams()
    if "needs_layout_passes" in pltpu.CompilerParams.__dataclass_fields__:
        cp = dataclasses.replace(cp, needs_layout_passes=False)
    mesh = plsc.VectorSubcoreMesh(core_axis_name="c", subcore_axis_name="s")
    k = pl.kernel(
        _sc_kernel,
        out_type=jax.ShapeDtypeStruct((B, H, N, T), jnp.float32),
        mesh=mesh,
        compiler_params=cp,
        scratch_types=[
            pltpu.VMEM((TCH * N,), jnp.int32),
            pltpu.VMEM((N * TS,), jnp.int32),
            pltpu.VMEM((TCH * NFC,), jnp.float32),
            pltpu.VMEM((NFC * TS,), jnp.float32),
            pltpu.VMEM((NAA * HE * L,), jnp.float32),
            pltpu.VMEM((HG, N, TCH), jnp.float32),
            pltpu.VMEM((HG, N, TCH), jnp.float32),
            pltpu.SemaphoreType.DMA,
            pltpu.SemaphoreType.DMA,
        ],
    )
    return k(x2, f2, lut)
